# baseline (speedup 1.0000x reference)
import jax
import jax.numpy as jnp
from jax import lax
from jax.experimental import pallas as pl
from jax.experimental.pallas import tpu as pltpu
from jax.experimental.pallas import tpu_sc as plsc

_NC = 2
_NS = 16
_NW = _NC * _NS
_CHUNK = 512
_D = 64


def _emb_body(idx_hbm, table_hbm, out_hbm, idx_v, rows0, rows1, gs0, gs1, ws0, ws1):
    wid = lax.axis_index("s") * _NC + lax.axis_index("c")
    n_per_w = idx_v.shape[0]
    n_groups = n_per_w // _CHUNK
    base = wid * n_per_w
    pltpu.sync_copy(idx_hbm.at[pl.ds(base, n_per_w)], idx_v)

    def fire(g, rows, gsem):
        pltpu.async_copy(
            table_hbm.at[idx_v.at[pl.ds(g * _CHUNK, _CHUNK)]], rows, gsem)

    def drain_gather(rows, gsem):
        pltpu.make_async_copy(
            table_hbm.at[idx_v.at[pl.ds(0, _CHUNK)]], rows, gsem).wait()

    def write(g, rows, wsem):
        pltpu.async_copy(
            rows, out_hbm.at[pl.ds(base + g * _CHUNK, _CHUNK)], wsem)

    def wait_write(rows, wsem):
        pltpu.make_async_copy(rows, out_hbm.at[pl.ds(base, _CHUNK)], wsem).wait()

    fire(0, rows0, gs0)
    last = n_groups - 1

    def pair(p, carry):
        r_odd = 2 * p + 1

        @pl.when(p >= 1)
        def _():
            wait_write(rows1, ws1)
        fire(r_odd, rows1, gs1)
        drain_gather(rows0, gs0)
        write(r_odd - 1, rows0, ws0)

        wait_write(rows0, ws0)
        fire(r_odd + 1, rows0, gs0)
        drain_gather(rows1, gs1)
        write(r_odd, rows1, ws1)
        return carry

    lax.fori_loop(0, (n_groups - 2) // 2, pair, 0)
    wait_write(rows1, ws1)
    fire(last, rows1, gs1)
    drain_gather(rows0, gs0)
    write(last - 1, rows0, ws0)
    drain_gather(rows1, gs1)
    write(last, rows1, ws1)
    wait_write(rows0, ws0)
    wait_write(rows1, ws1)


def kernel(tokens, weight):
    s0, s1 = tokens.shape
    b = s0 * s1
    idx = tokens.reshape(b).astype(jnp.int32)
    mesh = plsc.VectorSubcoreMesh(core_axis_name="c", subcore_axis_name="s")
    out = pl.kernel(
        _emb_body,
        out_type=jax.ShapeDtypeStruct((b, _D), jnp.float32),
        mesh=mesh,
        compiler_params=pltpu.CompilerParams(use_tc_tiling_on_sc=False),
        scratch_types=[
            pltpu.VMEM((b // _NW,), jnp.int32),
            pltpu.VMEM((_CHUNK, _D), jnp.float32),
            pltpu.VMEM((_CHUNK, _D), jnp.float32),
            pltpu.SemaphoreType.DMA,
            pltpu.SemaphoreType.DMA,
            pltpu.SemaphoreType.DMA,
            pltpu.SemaphoreType.DMA,
        ],
    )(idx, weight)
    return out.reshape(s0, s1, _D)
